# baseline (speedup 1.0000x reference)
import jax
import jax.numpy as jnp
from jax.experimental import pallas as pl
from jax.experimental.pallas import tpu as pltpu

_NE = 1024
_D = 64
_R = 1024
_NT = 16 * 32 * 32
_G = _NT // _R


def _vq_body(x3_ref, xlin_ref, cb_ref, a2_ref, b2_ref, rcol_ref,
             idx_ref, qst_ref, loss_ref):
    g = pl.program_id(0)
    xbT = x3_ref[0]
    cb = cb_ref[...]

    abT = jax.lax.dot_general(cb, xbT, (((1,), (0,)), ((), ())),
                              preferred_element_type=jnp.float32)
    d2 = (a2_ref[...] + b2_ref[...]) - 2.0 * abT
    dist = jnp.sqrt(jnp.maximum(d2, 0.0))
    m = jnp.min(dist, axis=0, keepdims=True)
    cnd = jnp.where(dist == m, rcol_ref[...], jnp.float32(_NE))
    idxf = jnp.min(cnd, axis=0)
    idx_ref[0, 0, :] = idxf.astype(jnp.int32)

    encT = (cnd == idxf[None, :]).astype(jnp.float32)
    q = jax.lax.dot_general(encT, cb, (((0,), (0,)), ((), ())),
                            preferred_element_type=jnp.float32)

    xl = xlin_ref[...]
    diff = q - xl
    qst_ref[...] = xl + diff

    @pl.when(g == 0)
    def _init():
        loss_ref[0, 0] = 0.0
    loss_ref[0, 0] += jnp.sum(diff * diff)


def kernel(x, codebook):
    B, C, H, W = x.shape
    x3 = x.reshape(_G, _D, _R)
    xlin = x.reshape(_NT, _D)
    xt2 = jnp.transpose(x, (0, 2, 3, 1)).reshape(_NT, _D)
    a2 = jnp.sum(xt2 * xt2, axis=1)[None, :]
    b2 = jnp.sum(codebook * codebook, axis=1)[:, None]
    rcol = jnp.arange(_NE, dtype=jnp.float32)[:, None]

    idx3, qst2, loss_acc = pl.pallas_call(
        _vq_body,
        grid=(_G,),
        in_specs=[
            pl.BlockSpec((1, _D, _R), lambda g: (g, 0, 0)),
            pl.BlockSpec((_R, _D), lambda g: (g, 0)),
            pl.BlockSpec((_NE, _D), lambda g: (0, 0)),
            pl.BlockSpec((1, _R), lambda g: (0, g)),
            pl.BlockSpec((_NE, 1), lambda g: (0, 0)),
            pl.BlockSpec((_NE, 1), lambda g: (0, 0)),
        ],
        out_specs=[
            pl.BlockSpec((1, 1, _R), lambda g: (g, 0, 0)),
            pl.BlockSpec((_R, _D), lambda g: (g, 0)),
            pl.BlockSpec(memory_space=pltpu.SMEM, block_shape=(1, 1),
                         index_map=lambda g: (0, 0)),
        ],
        out_shape=[
            jax.ShapeDtypeStruct((_G, 1, _R), jnp.int32),
            jax.ShapeDtypeStruct((_NT, _D), jnp.float32),
            jax.ShapeDtypeStruct((1, 1), jnp.float32),
        ],
    )(x3, xlin, codebook, a2, b2, rcol)

    quantized_st = qst2.reshape(B, C, H, W)
    m = loss_acc[0, 0] / jnp.float32(_NT * _D)
    loss = m * jnp.float32(0.25) + m
    indices = idx3.reshape(B, H, W)
    return quantized_st, loss, indices


# speedup vs baseline: 1.3885x; 1.3885x over previous
import jax
import jax.numpy as jnp
from jax.experimental import pallas as pl
from jax.experimental.pallas import tpu as pltpu

_NE = 1024
_D = 64
_R = 1024
_NT = 16 * 32 * 32
_G = _NT // _R
_GS = _R // _D


def _vq_body(x3_ref, cb_ref, a2_ref, b2_ref, rcol_ref,
             idx_ref, qst_ref, loss_ref):
    g = pl.program_id(0)
    xbT = x3_ref[0]
    cb = cb_ref[...]

    abT = jax.lax.dot_general(cb, xbT, (((1,), (0,)), ((), ())),
                              preferred_element_type=jnp.float32)
    d2 = (a2_ref[...] + b2_ref[...]) - 2.0 * abT
    dist = jnp.sqrt(jnp.maximum(d2, 0.0))
    m = jnp.min(dist, axis=0, keepdims=True)
    cnd = jnp.where(dist == m, rcol_ref[...], jnp.float32(_NE))
    idxf = jnp.min(cnd, axis=0)
    idx_ref[0, 0, :] = idxf.astype(jnp.int32)

    encT = (cnd == idxf[None, :]).astype(jnp.float32)
    q = jax.lax.dot_general(encT, cb, (((0,), (0,)), ((), ())),
                            preferred_element_type=jnp.float32)
    q4 = q.reshape(_D, _GS, _D)

    acc = jnp.float32(0.0)
    for gs in range(_GS):
        xg = xbT[:, gs * _D:(gs + 1) * _D]
        dg = q4[:, gs, :] - xg
        qst_ref[0, :, gs * _D:(gs + 1) * _D] = xg + dg
        acc = acc + jnp.sum(dg * dg)

    @pl.when(g == 0)
    def _init():
        loss_ref[0, 0] = 0.0
    loss_ref[0, 0] += acc


def kernel(x, codebook):
    B, C, H, W = x.shape
    x3 = x.reshape(_G, _D, _R)
    xt2 = jnp.transpose(x, (0, 2, 3, 1)).reshape(_NT, _D)
    a2 = jnp.sum(xt2 * xt2, axis=1)[None, :]
    b2 = jnp.sum(codebook * codebook, axis=1)[:, None]
    rcol = jnp.arange(_NE, dtype=jnp.float32)[:, None]

    idx3, qst3, loss_acc = pl.pallas_call(
        _vq_body,
        grid=(_G,),
        in_specs=[
            pl.BlockSpec((1, _D, _R), lambda g: (g, 0, 0)),
            pl.BlockSpec((_NE, _D), lambda g: (0, 0)),
            pl.BlockSpec((1, _R), lambda g: (0, g)),
            pl.BlockSpec((_NE, 1), lambda g: (0, 0)),
            pl.BlockSpec((_NE, 1), lambda g: (0, 0)),
        ],
        out_specs=[
            pl.BlockSpec((1, 1, _R), lambda g: (g, 0, 0)),
            pl.BlockSpec((1, _D, _R), lambda g: (g, 0, 0)),
            pl.BlockSpec(memory_space=pltpu.SMEM, block_shape=(1, 1),
                         index_map=lambda g: (0, 0)),
        ],
        out_shape=[
            jax.ShapeDtypeStruct((_G, 1, _R), jnp.int32),
            jax.ShapeDtypeStruct((_G, _D, _R), jnp.float32),
            jax.ShapeDtypeStruct((1, 1), jnp.float32),
        ],
    )(x3, codebook, a2, b2, rcol)

    quantized_st = qst3.reshape(B, C, H, W)
    m = loss_acc[0, 0] / jnp.float32(_NT * _D)
    loss = m * jnp.float32(0.25) + m
    indices = idx3.reshape(B, H, W)
    return quantized_st, loss, indices


# speedup vs baseline: 1.4680x; 1.0573x over previous
import jax
import jax.numpy as jnp
from jax.experimental import pallas as pl
from jax.experimental.pallas import tpu as pltpu

_NE = 1024
_D = 64
_R = 1024
_NT = 16 * 32 * 32
_G = _NT // _R
_GS = _R // _D


def _vq_body(x3_ref, cb_ref, a2_ref, b2_ref, rcol_ref,
             idx_ref, qst_ref, loss_ref):
    g = pl.program_id(0)
    xbT = x3_ref[0]
    cb = cb_ref[...]

    abT = jax.lax.dot_general(cb, xbT, (((1,), (0,)), ((), ())),
                              preferred_element_type=jnp.float32)
    d2 = (a2_ref[...] + b2_ref[...]) - 2.0 * abT
    dist = jnp.sqrt(jnp.maximum(d2, 0.0))
    m = jnp.min(dist, axis=0, keepdims=True)
    cnd = jnp.where(dist == m, rcol_ref[...], jnp.float32(_NE))
    idxf = jnp.min(cnd, axis=0)
    idx_ref[0, 0, :] = idxf.astype(jnp.int32)

    encT = (cnd == idxf[None, :]).astype(jnp.float32)
    q = jax.lax.dot_general(encT, cb, (((0,), (0,)), ((), ())),
                            preferred_element_type=jnp.float32)
    q4 = q.reshape(_D, _GS, _D)

    dmis = jnp.concatenate(
        [q4[:, gs, :] - xbT[:, gs * _D:(gs + 1) * _D] for gs in range(_GS)],
        axis=1)
    qst_ref[0] = xbT + dmis
    acc = jnp.sum(dmis * dmis)

    @pl.when(g == 0)
    def _init():
        loss_ref[0, 0] = 0.0
    loss_ref[0, 0] += acc


def kernel(x, codebook):
    B, C, H, W = x.shape
    x3 = x.reshape(_G, _D, _R)
    xt2 = jnp.transpose(x, (0, 2, 3, 1)).reshape(_NT, _D)
    a2 = jnp.sum(xt2 * xt2, axis=1)[None, :]
    b2 = jnp.sum(codebook * codebook, axis=1)[:, None]
    rcol = jnp.arange(_NE, dtype=jnp.float32)[:, None]

    idx3, qst3, loss_acc = pl.pallas_call(
        _vq_body,
        grid=(_G,),
        in_specs=[
            pl.BlockSpec((1, _D, _R), lambda g: (g, 0, 0)),
            pl.BlockSpec((_NE, _D), lambda g: (0, 0)),
            pl.BlockSpec((1, _R), lambda g: (0, g)),
            pl.BlockSpec((_NE, 1), lambda g: (0, 0)),
            pl.BlockSpec((_NE, 1), lambda g: (0, 0)),
        ],
        out_specs=[
            pl.BlockSpec((1, 1, _R), lambda g: (g, 0, 0)),
            pl.BlockSpec((1, _D, _R), lambda g: (g, 0, 0)),
            pl.BlockSpec(memory_space=pltpu.SMEM, block_shape=(1, 1),
                         index_map=lambda g: (0, 0)),
        ],
        out_shape=[
            jax.ShapeDtypeStruct((_G, 1, _R), jnp.int32),
            jax.ShapeDtypeStruct((_G, _D, _R), jnp.float32),
            jax.ShapeDtypeStruct((1, 1), jnp.float32),
        ],
    )(x3, codebook, a2, b2, rcol)

    quantized_st = qst3.reshape(B, C, H, W)
    m = loss_acc[0, 0] / jnp.float32(_NT * _D)
    loss = m * jnp.float32(0.25) + m
    indices = idx3.reshape(B, H, W)
    return quantized_st, loss, indices
